# baseline (speedup 1.0000x reference)
import jax
import jax.numpy as jnp
from jax.experimental import pallas as pl

DIM = 2048
N_EXPERTS = 64
TOPK = 8
N_GROUPS = 8
EPG = N_EXPERTS // N_GROUPS
TOPK_GROUPS = 4
ROUTE_SCALE = 2.5
BLK = 512
CHUNK = 64

_NEG = -1e30


def _rotl(v, d):
    return jnp.concatenate([v[:, d:], v[:, :d]], axis=1)


def _rotr(v, d):
    return jnp.concatenate([v[:, -d:], v[:, :-d]], axis=1)


def _route(x, W, b, lane, gid):
    logits = jax.lax.dot_general(
        x, W, (((1,), (1,)), ((), ())), preferred_element_type=jnp.float32
    )
    s = jax.nn.sigmoid(logits)
    sb = s + b

    h2 = _rotl(sb, 4)
    hi = jnp.maximum(sb, h2)
    lo = jnp.minimum(sb, h2)
    for d in (2, 1):
        hi2 = _rotl(hi, d)
        lo2 = _rotl(lo, d)
        nlo = jnp.maximum(jnp.minimum(hi, hi2), jnp.where(hi >= hi2, lo, lo2))
        hi = jnp.maximum(hi, hi2)
        lo = nlo

    gsm = jnp.where(lane % EPG == 0, hi + lo, _NEG)

    keepf = jnp.zeros_like(sb)
    for t in range(TOPK_GROUPS):
        am = jnp.argmax(gsm, axis=1, keepdims=True)
        sel = gid == (am // EPG)
        keepf = jnp.where(sel, 1.0, keepf)
        if t < TOPK_GROUPS - 1:
            gsm = jnp.where(sel, _NEG, gsm)

    masked = sb * keepf

    idx_cols = []
    w_cols = []
    for t in range(TOPK):
        am = jnp.argmax(masked, axis=1, keepdims=True)
        onehot = lane == am
        w_cols.append(jnp.sum(jnp.where(onehot, s, 0.0), axis=1, keepdims=True))
        idx_cols.append(am.astype(jnp.int32))
        if t < TOPK - 1:
            masked = jnp.where(onehot, _NEG, masked)

    wsum = w_cols[0]
    for c in w_cols[1:]:
        wsum = wsum + c
    scale = ROUTE_SCALE / wsum
    return [c * scale for c in w_cols], idx_cols


def _gate_kernel(x_ref, w_ref, b_ref, wout_ref, iout_ref):
    W = w_ref[...]
    b = b_ref[...]
    lane = jax.lax.broadcasted_iota(jnp.int32, (1, N_EXPERTS), 1)
    gid = lane // EPG

    outs = [
        _route(x_ref[r:r + CHUNK, :], W, b, lane, gid)
        for r in range(0, BLK, CHUNK)
    ]
    for i, (w_cols, idx_cols) in enumerate(outs):
        r0 = i * CHUNK
        for k in range(TOPK):
            wout_ref[r0:r0 + CHUNK, k:k + 1] = w_cols[k]
            iout_ref[r0:r0 + CHUNK, k:k + 1] = idx_cols[k]


def kernel(x, W, b):
    B = x.shape[0]
    b2 = b.reshape(1, N_EXPERTS)
    grid = (B // BLK,)
    wts, idxs = pl.pallas_call(
        _gate_kernel,
        grid=grid,
        in_specs=[
            pl.BlockSpec((BLK, DIM), lambda i: (i, 0)),
            pl.BlockSpec((N_EXPERTS, DIM), lambda i: (0, 0)),
            pl.BlockSpec((1, N_EXPERTS), lambda i: (0, 0)),
        ],
        out_specs=[
            pl.BlockSpec((BLK, TOPK), lambda i: (i, 0)),
            pl.BlockSpec((BLK, TOPK), lambda i: (i, 0)),
        ],
        out_shape=[
            jax.ShapeDtypeStruct((B, TOPK), jnp.float32),
            jax.ShapeDtypeStruct((B, TOPK), jnp.int32),
        ],
    )(x, W, b2)
    return wts.astype(x.dtype), idxs


# speedup vs baseline: 1.1267x; 1.1267x over previous
import jax
import jax.numpy as jnp
from jax.experimental import pallas as pl

DIM = 2048
N_EXPERTS = 64
TOPK = 8
N_GROUPS = 8
EPG = N_EXPERTS // N_GROUPS
TOPK_GROUPS = 4
ROUTE_SCALE = 2.5
BLK = 512
CHUNK = 256

_NEG = -1e30


def _rotl(v, d):
    return jnp.concatenate([v[:, d:], v[:, :d]], axis=1)


def _rotr(v, d):
    return jnp.concatenate([v[:, -d:], v[:, :-d]], axis=1)


def _route(x, W, b, lane, gid):
    logits = jax.lax.dot_general(
        x, W, (((1,), (1,)), ((), ())), preferred_element_type=jnp.float32
    )
    s = jax.nn.sigmoid(logits)
    sb = s + b

    h2 = _rotl(sb, 4)
    hi = jnp.maximum(sb, h2)
    lo = jnp.minimum(sb, h2)
    for d in (2, 1):
        hi2 = _rotl(hi, d)
        lo2 = _rotl(lo, d)
        nlo = jnp.maximum(jnp.minimum(hi, hi2), jnp.where(hi >= hi2, lo, lo2))
        hi = jnp.maximum(hi, hi2)
        lo = nlo

    gsm = jnp.where(lane % EPG == 0, hi + lo, _NEG)

    keepf = jnp.zeros_like(sb)
    for t in range(TOPK_GROUPS):
        am = jnp.argmax(gsm, axis=1, keepdims=True)
        sel = gid == (am // EPG)
        keepf = jnp.where(sel, 1.0, keepf)
        if t < TOPK_GROUPS - 1:
            gsm = jnp.where(sel, _NEG, gsm)

    masked = sb * keepf

    idx_cols = []
    w_cols = []
    for t in range(TOPK):
        am = jnp.argmax(masked, axis=1, keepdims=True)
        onehot = lane == am
        w_cols.append(jnp.sum(jnp.where(onehot, s, 0.0), axis=1, keepdims=True))
        idx_cols.append(am.astype(jnp.int32))
        if t < TOPK - 1:
            masked = jnp.where(onehot, _NEG, masked)

    wsum = w_cols[0]
    for c in w_cols[1:]:
        wsum = wsum + c
    scale = ROUTE_SCALE / wsum
    return [c * scale for c in w_cols], idx_cols


def _gate_kernel(x_ref, w_ref, b_ref, wout_ref, iout_ref):
    W = w_ref[...]
    b = b_ref[...]
    lane = jax.lax.broadcasted_iota(jnp.int32, (1, N_EXPERTS), 1)
    gid = lane // EPG

    outs = [
        _route(x_ref[r:r + CHUNK, :], W, b, lane, gid)
        for r in range(0, BLK, CHUNK)
    ]
    for i, (w_cols, idx_cols) in enumerate(outs):
        r0 = i * CHUNK
        for k in range(TOPK):
            wout_ref[r0:r0 + CHUNK, k:k + 1] = w_cols[k]
            iout_ref[r0:r0 + CHUNK, k:k + 1] = idx_cols[k]


def kernel(x, W, b):
    B = x.shape[0]
    b2 = b.reshape(1, N_EXPERTS)
    grid = (B // BLK,)
    wts, idxs = pl.pallas_call(
        _gate_kernel,
        grid=grid,
        in_specs=[
            pl.BlockSpec((BLK, DIM), lambda i: (i, 0)),
            pl.BlockSpec((N_EXPERTS, DIM), lambda i: (0, 0)),
            pl.BlockSpec((1, N_EXPERTS), lambda i: (0, 0)),
        ],
        out_specs=[
            pl.BlockSpec((BLK, TOPK), lambda i: (i, 0)),
            pl.BlockSpec((BLK, TOPK), lambda i: (i, 0)),
        ],
        out_shape=[
            jax.ShapeDtypeStruct((B, TOPK), jnp.float32),
            jax.ShapeDtypeStruct((B, TOPK), jnp.int32),
        ],
    )(x, W, b2)
    return wts.astype(x.dtype), idxs
